# baseline (speedup 1.0000x reference)
import functools

import jax
import jax.numpy as jnp
from jax import lax
from jax.experimental import pallas as pl
from jax.experimental.pallas import tpu as pltpu
from jax.experimental.pallas import tpu_sc as plsc

R, C = 128, 100000
L = 16
NC, NS = 2, 16
NW = NC * NS
ROWS_PER_W = R // NW
CHUNKS = C // L


def _softmax_body(in_hbm, out_hbm, row_v):
    c = lax.axis_index("c")
    s = lax.axis_index("s")
    wid = s * NC + c
    base = wid * ROWS_PER_W

    for r in range(ROWS_PER_W):
        row = base + r
        pltpu.sync_copy(in_hbm.at[row], row_v)

        def max_body(i, m):
            return jnp.maximum(m, row_v[pl.ds(i * L, L)])

        m = lax.fori_loop(0, CHUNKS, max_body,
                          jnp.full((L,), -3.0e38, jnp.float32))
        mv = jnp.full((L,), jnp.max(m), jnp.float32)

        def exp_body(i, acc):
            e = jnp.exp(row_v[pl.ds(i * L, L)] - mv)
            row_v[pl.ds(i * L, L)] = e
            return acc + e

        ssum = lax.fori_loop(0, CHUNKS, exp_body,
                             jnp.zeros((L,), jnp.float32))
        sv = jnp.full((L,), jnp.sum(ssum), jnp.float32)
        iv = jnp.ones((L,), jnp.float32) / sv

        def div_body(i, carry):
            row_v[pl.ds(i * L, L)] = row_v[pl.ds(i * L, L)] * iv
            return carry

        lax.fori_loop(0, CHUNKS, div_body, 0)
        pltpu.sync_copy(row_v, out_hbm.at[row])


@jax.jit
def kernel(inputs):
    run = functools.partial(
        pl.kernel,
        out_type=jax.ShapeDtypeStruct((R, C), jnp.float32),
        mesh=plsc.VectorSubcoreMesh(core_axis_name="c", subcore_axis_name="s"),
        scratch_types=[pltpu.VMEM((C,), jnp.float32)],
        compiler_params=pltpu.CompilerParams(needs_layout_passes=False),
    )(_softmax_body)
    return run(inputs)


# speedup vs baseline: 2.5454x; 2.5454x over previous
import functools

import jax
import jax.numpy as jnp
from jax import lax
from jax.experimental import pallas as pl
from jax.experimental.pallas import tpu as pltpu
from jax.experimental.pallas import tpu_sc as plsc

R, C = 128, 100000
L = 16
NC, NS = 2, 16
NW = NC * NS
ROWS_PER_W = R // NW
CHUNKS = C // L


U = 10
A = 5
STEPS = CHUNKS // U


def _softmax_body(in_hbm, out_hbm, row_v):
    c = lax.axis_index("c")
    s = lax.axis_index("s")
    wid = s * NC + c
    base = wid * ROWS_PER_W

    for r in range(ROWS_PER_W):
        row = base + r
        pltpu.sync_copy(in_hbm.at[row], row_v)

        def max_body(i, ms):
            ms = list(ms)
            for u in range(U):
                x = row_v[pl.ds((i * U + u) * L, L)]
                ms[u % A] = jnp.maximum(ms[u % A], x)
            return tuple(ms)

        init_m = tuple(jnp.full((L,), -3.0e38, jnp.float32)
                       for _ in range(A))
        ms = lax.fori_loop(0, STEPS, max_body, init_m)
        m = ms[0]
        for a in range(1, A):
            m = jnp.maximum(m, ms[a])
        mv = jnp.full((L,), jnp.max(m), jnp.float32)

        def exp_body(i, accs):
            accs = list(accs)
            for u in range(U):
                sl = pl.ds((i * U + u) * L, L)
                e = jnp.exp(row_v[sl] - mv)
                row_v[sl] = e
                accs[u % A] = accs[u % A] + e
            return tuple(accs)

        init_s = tuple(jnp.zeros((L,), jnp.float32) for _ in range(A))
        accs = lax.fori_loop(0, STEPS, exp_body, init_s)
        ssum = accs[0]
        for a in range(1, A):
            ssum = ssum + accs[a]
        sv = jnp.full((L,), jnp.sum(ssum), jnp.float32)
        iv = jnp.ones((L,), jnp.float32) / sv

        def div_body(i, carry):
            for u in range(U):
                sl = pl.ds((i * U + u) * L, L)
                row_v[sl] = row_v[sl] * iv
            return carry

        lax.fori_loop(0, STEPS, div_body, 0)
        pltpu.sync_copy(row_v, out_hbm.at[row])


@jax.jit
def kernel(inputs):
    run = functools.partial(
        pl.kernel,
        out_type=jax.ShapeDtypeStruct((R, C), jnp.float32),
        mesh=plsc.VectorSubcoreMesh(core_axis_name="c", subcore_axis_name="s"),
        scratch_types=[pltpu.VMEM((C,), jnp.float32)],
        compiler_params=pltpu.CompilerParams(needs_layout_passes=False),
    )(_softmax_body)
    return run(inputs)
